# baseline (speedup 1.0000x reference)
import functools

import jax
import jax.numpy as jnp
from jax import lax
from jax.experimental import pallas as pl
from jax.experimental.pallas import tpu as pltpu
from jax.experimental.pallas import tpu_sc as plsc

N_NODES = 10000
N_EDGES = 320000
D = 128

NC = 2
NS = 16
NW = NC * NS
EPW = N_EDGES // NW
CH = 80
NFULL = EPW // CH
ROWS_PT = 624
ROWS_LAST = N_NODES - (NS - 1) * ROWS_PT
CNT_PT = 624
NB = 2


@functools.cache
def _make_seg_sum(with_cnt: bool):
  mesh = plsc.VectorSubcoreMesh(
      core_axis_name="c", subcore_axis_name="s", num_cores=NC,
      num_subcores=NS)

  out_type = [jax.ShapeDtypeStruct((NC, N_NODES, D), jnp.float32)]
  if with_cnt:
    out_type.append(jax.ShapeDtypeStruct((NC * N_NODES,), jnp.float32))

  scratch = dict(
      sidx=pltpu.VMEM((EPW,), jnp.int32),
      didx=pltpu.VMEM((EPW,), jnp.int32),
      rows=[pltpu.VMEM((CH, D), jnp.float32) for _ in range(NB)],
      sbuf=[pltpu.VMEM((CH,), jnp.int32) for _ in range(NB)],
      dbuf=[pltpu.VMEM((CH,), jnp.int32) for _ in range(NB)],
      ones_v=pltpu.VMEM((CH,), jnp.float32),
      cbuf=pltpu.VMEM((ROWS_LAST,), jnp.float32),
      acc_sp=pltpu.VMEM_SHARED((N_NODES, D), jnp.float32),
      cnt_sp=pltpu.VMEM_SHARED((N_NODES,), jnp.float32),
      sem_g=[pltpu.SemaphoreType.DMA for _ in range(NB)],
      sem_s=[pltpu.SemaphoreType.DMA for _ in range(NB)],
      sem_c=[pltpu.SemaphoreType.DMA for _ in range(NB)],
  )

  def body(src_hbm, dst_hbm, feat_hbm, z2d_hbm, *outs, sidx, didx, rows,
           sbuf, dbuf, ones_v, cbuf, acc_sp, cnt_sp, sem_g, sem_s, sem_c):
    if with_cnt:
      acc_out, cnt_out = outs
    else:
      (acc_out,) = outs

    cid = lax.axis_index("c")
    sid = lax.axis_index("s")
    wid = cid * NS + sid

    IC = 2000

    def idx_stage(p, _):
      off = p * IC
      pltpu.sync_copy(src_hbm.at[pl.ds(wid * EPW + off, IC)],
                      sidx.at[pl.ds(off, IC)])
      pltpu.sync_copy(dst_hbm.at[pl.ds(wid * EPW + off, IC)],
                      didx.at[pl.ds(off, IC)])
      return 0

    lax.fori_loop(0, EPW // IC, idx_stage, 0)

    @pl.when(sid < NS - 1)
    def _():
      pltpu.sync_copy(z2d_hbm.at[pl.ds(sid * ROWS_PT, ROWS_PT)],
                      acc_sp.at[pl.ds(sid * ROWS_PT, ROWS_PT)])

    @pl.when(sid == NS - 1)
    def _():
      pltpu.sync_copy(z2d_hbm.at[pl.ds((NS - 1) * ROWS_PT, ROWS_LAST)],
                      acc_sp.at[pl.ds((NS - 1) * ROWS_PT, ROWS_LAST)])

    if with_cnt:
      for j in range(CH // 16):
        ones_v[pl.ds(j * 16, 16)] = jnp.ones((16,), jnp.float32)
      for j in range(ROWS_LAST // 16):
        cbuf[pl.ds(j * 16, 16)] = jnp.zeros((16,), jnp.float32)

      @pl.when(sid < NS - 1)
      def _():
        pltpu.sync_copy(cbuf.at[pl.ds(0, CNT_PT)],
                        cnt_sp.at[pl.ds(sid * CNT_PT, CNT_PT)])

      @pl.when(sid == NS - 1)
      def _():
        nlast = N_NODES - (NS - 1) * CNT_PT
        pltpu.sync_copy(cbuf.at[pl.ds(0, nlast)],
                        cnt_sp.at[pl.ds((NS - 1) * CNT_PT, nlast)])

    plsc.subcore_barrier()

    def row(i):
      return jnp.minimum(i, NFULL - 1)

    def gather_start(i, b):
      base = row(i) * CH
      for k in range(CH // 16):
        sbuf[b][pl.ds(k * 16, 16)] = sidx[pl.ds(base + k * 16, 16)]
      pltpu.async_copy(feat_hbm.at[sbuf[b]], rows[b], sem_g[b])

    def gather_wait(b):
      pltpu.make_async_copy(feat_hbm.at[sbuf[b]], rows[b], sem_g[b]).wait()

    def scatter_start(i, b):
      for k in range(CH // 16):
        dbuf[b][pl.ds(k * 16, 16)] = didx[pl.ds(i * CH + k * 16, 16)]
      pltpu.async_copy(rows[b], acc_sp.at[dbuf[b]], sem_s[b], add=True)
      if with_cnt:
        pltpu.async_copy(ones_v, cnt_sp.at[dbuf[b]], sem_c[b], add=True)

    def scatter_wait(b):
      pltpu.make_async_copy(rows[b], acc_sp.at[dbuf[b]], sem_s[b]).wait()
      if with_cnt:
        pltpu.make_async_copy(ones_v, cnt_sp.at[dbuf[b]], sem_c[b]).wait()

    def step(i, b, first=False):
      nb = (b + 1) % NB
      if not first:
        scatter_wait(nb)
      gather_start(i + 1, nb)
      gather_wait(b)
      scatter_start(i, b)

    gather_start(0, 0)
    step(0, 0, first=True)

    def group(j, _):
      i0 = NB * j + 1
      for k in range(NB):
        step(i0 + k, (1 + k) % NB)
      return 0

    n_grp = (NFULL - 1) // NB
    lax.fori_loop(0, n_grp, group, 0)

    for i in range(1 + NB * n_grp, NFULL):
      step(i, i % NB)

    scatter_wait((NFULL - 1) % NB)
    gather_wait(NFULL % NB)

    plsc.subcore_barrier()

    @pl.when(sid < NS - 1)
    def _():
      pltpu.sync_copy(acc_sp.at[pl.ds(sid * ROWS_PT, ROWS_PT)],
                      acc_out.at[cid, pl.ds(sid * ROWS_PT, ROWS_PT)])

    @pl.when(sid == NS - 1)
    def _():
      pltpu.sync_copy(acc_sp.at[pl.ds((NS - 1) * ROWS_PT, ROWS_LAST)],
                      acc_out.at[cid, pl.ds((NS - 1) * ROWS_PT, ROWS_LAST)])

    if with_cnt:
      @pl.when(sid < NS - 1)
      def _():
        pltpu.sync_copy(cnt_sp.at[pl.ds(sid * CNT_PT, CNT_PT)],
                        cbuf.at[pl.ds(0, CNT_PT)])
        pltpu.sync_copy(
            cbuf.at[pl.ds(0, CNT_PT)],
            cnt_out.at[pl.ds(cid * N_NODES + sid * CNT_PT, CNT_PT)])

      @pl.when(sid == NS - 1)
      def _():
        nlast = N_NODES - (NS - 1) * CNT_PT
        pltpu.sync_copy(cnt_sp.at[pl.ds((NS - 1) * CNT_PT, nlast)],
                        cbuf.at[pl.ds(0, nlast)])
        pltpu.sync_copy(
            cbuf.at[pl.ds(0, nlast)],
            cnt_out.at[pl.ds(cid * N_NODES + (NS - 1) * CNT_PT, nlast)])

  return pl.kernel(body, out_type=out_type, mesh=mesh,
                   scratch_types=scratch,
                   name="seg_sum_cnt" if with_cnt else "seg_sum")


RB = 1000
GRID = N_NODES // RB


def _mm_bias_body(x_ref, w_ref, b_ref, o_ref):
  o_ref[...] = (jnp.dot(x_ref[...], w_ref[...],
                        preferred_element_type=jnp.float32) + b_ref[...])


def _mm_bias(x, w, b):
  return pl.pallas_call(
      _mm_bias_body,
      grid=(GRID,),
      in_specs=[
          pl.BlockSpec((RB, D), lambda i: (i, 0)),
          pl.BlockSpec((D, D), lambda i: (0, 0)),
          pl.BlockSpec((D,), lambda i: (0,)),
      ],
      out_specs=pl.BlockSpec((RB, D), lambda i: (i, 0)),
      out_shape=jax.ShapeDtypeStruct((N_NODES, D), jnp.float32),
  )(x, w, b)


def _combine_body(acc_ref, cnt_ref, xr_ref, wl_ref, o_ref, *, final: bool):
  s = acc_ref[0] + acc_ref[1]
  c = cnt_ref[0] + cnt_ref[1]
  mean = s / jnp.maximum(c, 1.0)
  z = (jnp.dot(mean, wl_ref[...], preferred_element_type=jnp.float32)
       + xr_ref[...])
  if final:
    m = jnp.max(z, axis=1, keepdims=True)
    e = jnp.exp(z - m)
    lse = jnp.log(jnp.sum(e, axis=1, keepdims=True)) + m
    o_ref[...] = z - lse
  else:
    o_ref[...] = jnp.maximum(z, 0.0)


def _combine(acc, cnt, xr, w_l, final: bool):
  return pl.pallas_call(
      functools.partial(_combine_body, final=final),
      grid=(GRID,),
      in_specs=[
          pl.BlockSpec((NC, RB, D), lambda i: (0, i, 0)),
          pl.BlockSpec((NC, RB, 1), lambda i: (0, i, 0)),
          pl.BlockSpec((RB, D), lambda i: (i, 0)),
          pl.BlockSpec((D, D), lambda i: (0, 0)),
      ],
      out_specs=pl.BlockSpec((RB, D), lambda i: (i, 0)),
      out_shape=jax.ShapeDtypeStruct((N_NODES, D), jnp.float32),
  )(acc, cnt, xr, w_l)


def kernel(x, edge_index, W1_l, W1_r, b1, W2_l, W2_r, b2):
  ei = edge_index.astype(jnp.int32)
  src = ei[0]
  dst = ei[1]
  z2d = jnp.zeros((N_NODES, D), jnp.float32)

  xr = _mm_bias(x, W1_r, b1)
  acc1, cnt = _make_seg_sum(True)(src, dst, x, z2d)
  cnt3 = cnt.reshape(NC, N_NODES, 1)
  h = _combine(acc1, cnt3, xr, W1_l, final=False)
  hr = _mm_bias(h, W2_r, b2)
  (acc2,) = _make_seg_sum(False)(src, dst, h, z2d)
  out = _combine(acc2, cnt3, hr, W2_l, final=True)
  return out
